# baseline (speedup 1.0000x reference)
import functools

import jax
import jax.numpy as jnp
from jax import lax
from jax.experimental import pallas as pl
from jax.experimental.pallas import tpu as pltpu
from jax.experimental.pallas import tpu_sc as plsc

NC, NS = 2, 16
NW = NC * NS
EPS = 1e-8
BN_EPS = 1e-5


def _to_bf16_linear(table, rows_per_block):
    n, e = table.shape

    def body(t_ref, o_ref):
        o_ref[...] = t_ref[...].astype(jnp.bfloat16)

    return pl.pallas_call(
        body,
        grid=(n // rows_per_block,),
        in_specs=[pl.BlockSpec((rows_per_block, e), lambda i: (i, 0))],
        out_specs=pl.BlockSpec((rows_per_block, e), lambda i: (i, 0)),
        out_shape=jax.ShapeDtypeStruct((n, e), jnp.bfloat16),
    )(table)


def _gather_rows(tab16, seqs):
    b, l = seqs.shape
    n, e = tab16.shape
    bpw = b // NW
    nbuf = 4
    mesh = plsc.VectorSubcoreMesh(core_axis_name="c", subcore_axis_name="s")

    @functools.partial(
        pl.kernel,
        out_type=jax.ShapeDtypeStruct((b, l, e), jnp.bfloat16),
        mesh=mesh,
        scratch_types=[
            pltpu.VMEM((bpw, l), jnp.int32),
            pltpu.VMEM((nbuf, l, e), jnp.bfloat16),
            pltpu.SemaphoreType.DMA((nbuf,)),
            pltpu.SemaphoreType.DMA((nbuf,)),
        ],
        compiler_params=pltpu.CompilerParams(use_tc_tiling_on_sc=False),
    )
    def kern(tab_hbm, seq_hbm, out_hbm, idx_v, rows_v, gsems, osems):
        wid = lax.axis_index("s") * NC + lax.axis_index("c")
        base = wid * bpw
        pltpu.sync_copy(seq_hbm.at[pl.ds(base, bpw)], idx_v)

        def fire(g, slot):
            pltpu.async_copy(
                tab_hbm.at[idx_v.at[g]], rows_v.at[slot], gsems.at[slot])

        def gwait(g, slot):
            pltpu.make_async_copy(
                tab_hbm.at[idx_v.at[g]], rows_v.at[slot], gsems.at[slot]).wait()

        def owait(g, slot):
            pltpu.make_async_copy(
                rows_v.at[slot], out_hbm.at[base + g], osems.at[slot]).wait()

        for p in range(nbuf - 1):
            fire(p, p)

        def body(g, carry):
            slot = lax.rem(g, nbuf)

            @pl.when(g + nbuf - 1 < bpw)
            def _():
                ns = lax.rem(g + nbuf - 1, nbuf)

                @pl.when(g >= 1)
                def _():
                    owait(g - 1, ns)

                fire(g + nbuf - 1, ns)

            gwait(g, slot)
            pltpu.async_copy(rows_v.at[slot], out_hbm.at[base + g],
                             osems.at[slot])
            return carry

        lax.fori_loop(0, bpw, body, 0)
        for t in range(nbuf):
            g = bpw - nbuf + t
            owait(g, lax.rem(g, nbuf))

    return kern(tab16, seqs)


def _emb_scores(emb3, w, bb):
    b, l2, ee = emb3.shape
    g = b // bb

    def body(e_ref, w_ref, se_ref, so_ref, me_ref, mo_ref, p_ref):
        e = e_ref[...].astype(jnp.float32)
        lo = e[:, :, :64]
        hi = e[:, :, 64:]
        wv = w_ref[0]
        dn = (((2,), (0,)), ((), ()))
        se = lax.dot_general(lo, wv, dn, preferred_element_type=jnp.float32)
        so = lax.dot_general(hi, wv, dn, preferred_element_type=jnp.float32)
        se_ref[...] = se
        so_ref[...] = so
        me_ref[...] = jnp.any(lo != 0.0, axis=2).astype(jnp.float32)
        mo_ref[...] = jnp.any(hi != 0.0, axis=2).astype(jnp.float32)
        parts = jnp.concatenate(
            [jnp.sum(se, axis=0, keepdims=True),
             jnp.sum(se * se, axis=0, keepdims=True),
             jnp.sum(so, axis=0, keepdims=True),
             jnp.sum(so * so, axis=0, keepdims=True)], axis=0)
        p_ref[...] = parts.reshape(1, 4, l2)

    return pl.pallas_call(
        body,
        grid=(g,),
        in_specs=[
            pl.BlockSpec((bb, l2, ee), lambda i: (i, 0, 0)),
            pl.BlockSpec((1, 64), lambda i: (0, 0)),
        ],
        out_specs=[
            pl.BlockSpec((bb, l2), lambda i: (i, 0)),
            pl.BlockSpec((bb, l2), lambda i: (i, 0)),
            pl.BlockSpec((bb, l2), lambda i: (i, 0)),
            pl.BlockSpec((bb, l2), lambda i: (i, 0)),
            pl.BlockSpec((1, 4, l2), lambda i: (i, 0, 0)),
        ],
        out_shape=[
            jax.ShapeDtypeStruct((b, l2), jnp.float32),
            jax.ShapeDtypeStruct((b, l2), jnp.float32),
            jax.ShapeDtypeStruct((b, l2), jnp.float32),
            jax.ShapeDtypeStruct((b, l2), jnp.float32),
            jax.ShapeDtypeStruct((g, 4, l2), jnp.float32),
        ],
    )(emb3, w.reshape(1, 64))


def _weights_apply(se, so, me, mo, parts, gamma_e, gamma_o, beta_e, beta_o,
                   batch, bb):
    b, l2 = se.shape
    g = b // bb
    npart = parts.shape[0]

    def body(se_ref, so_ref, me_ref, mo_ref, p_ref, ge_ref, go_ref, be_ref,
             bo_ref, we_ref, wo_ref):
        p = jnp.sum(p_ref[...], axis=0)
        mean_e = p[0:1] / batch
        msq_e = p[1:2] / batch
        mean_o = p[2:3] / batch
        msq_o = p[3:4] / batch
        var_e = msq_e - mean_e * mean_e
        var_o = msq_o - mean_o * mean_o
        xe = ge_ref[...] * (se_ref[...] - mean_e) / jnp.sqrt(var_e + BN_EPS) \
            + be_ref[...]
        xo = go_ref[...] * (so_ref[...] - mean_o) / jnp.sqrt(var_o + BN_EPS) \
            + bo_ref[...]
        we_ref[...] = jax.nn.sigmoid(xe) * me_ref[...]
        wo_ref[...] = jax.nn.sigmoid(xo) * mo_ref[...]

    return pl.pallas_call(
        body,
        grid=(g,),
        in_specs=[
            pl.BlockSpec((bb, l2), lambda i: (i, 0)),
            pl.BlockSpec((bb, l2), lambda i: (i, 0)),
            pl.BlockSpec((bb, l2), lambda i: (i, 0)),
            pl.BlockSpec((bb, l2), lambda i: (i, 0)),
            pl.BlockSpec((npart, 4, l2), lambda i: (0, 0, 0)),
            pl.BlockSpec((1, l2), lambda i: (0, 0)),
            pl.BlockSpec((1, l2), lambda i: (0, 0)),
            pl.BlockSpec((1, l2), lambda i: (0, 0)),
            pl.BlockSpec((1, l2), lambda i: (0, 0)),
        ],
        out_specs=[
            pl.BlockSpec((bb, l2), lambda i: (i, 0)),
            pl.BlockSpec((bb, l2), lambda i: (i, 0)),
        ],
        out_shape=[
            jax.ShapeDtypeStruct((b, l2), jnp.float32),
            jax.ShapeDtypeStruct((b, l2), jnp.float32),
        ],
    )(se, so, me, mo, parts, gamma_e.reshape(1, l2), gamma_o.reshape(1, l2),
      beta_e.reshape(1, l2), beta_o.reshape(1, l2))


def _pool_tc(emb3, we, wo, bb):
    b, l2, ee = emb3.shape
    g = b // bb

    def body(e_ref, we_ref, wo_ref, o_ref):
        e = e_ref[...].astype(jnp.float32)
        lo = e[:, :, :64]
        hi = e[:, :, 64:]
        we = we_ref[...]
        wo = wo_ref[...]
        dn = (((1,), (1,)), ((0,), (0,)))
        pe = lax.dot_general(we, lo, dn, preferred_element_type=jnp.float32)
        po = lax.dot_general(wo, hi, dn, preferred_element_type=jnp.float32)
        wsum = jnp.sum(we, axis=1, keepdims=True) \
            + jnp.sum(wo, axis=1, keepdims=True)
        o_ref[...] = (pe + po) / (wsum + EPS)

    return pl.pallas_call(
        body,
        grid=(g,),
        in_specs=[
            pl.BlockSpec((bb, l2, ee), lambda i: (i, 0, 0)),
            pl.BlockSpec((bb, l2), lambda i: (i, 0)),
            pl.BlockSpec((bb, l2), lambda i: (i, 0)),
        ],
        out_specs=pl.BlockSpec((bb, 64), lambda i: (i, 0)),
        out_shape=jax.ShapeDtypeStruct((b, 64), jnp.float32),
    )(emb3, we, wo)


def _similarity(pq, pc):
    b, e = pq.shape
    ti, tj = 256, 2048

    def body(q_ref, c_ref, o_ref):
        q = q_ref[...]
        c = c_ref[...]
        qn = q / (jnp.sqrt(jnp.sum(q * q, axis=1, keepdims=True)) + EPS)
        cn = c / (jnp.sqrt(jnp.sum(c * c, axis=1, keepdims=True)) + EPS)
        o_ref[...] = lax.dot_general(
            qn, cn, (((1,), (1,)), ((), ())),
            preferred_element_type=jnp.float32)

    return pl.pallas_call(
        body,
        grid=(b // ti, b // tj),
        in_specs=[
            pl.BlockSpec((ti, e), lambda i, j: (i, 0)),
            pl.BlockSpec((tj, e), lambda i, j: (j, 0)),
        ],
        out_specs=pl.BlockSpec((ti, tj), lambda i, j: (i, j)),
        out_shape=jax.ShapeDtypeStruct((b, b), jnp.float32),
    )(pq, pc)


def _encode(seqs, table, w, gamma, beta, rows_per_block):
    b, l = seqs.shape
    l2 = l // 2
    tab16 = _to_bf16_linear(table, rows_per_block)
    emb = _gather_rows(tab16, seqs)
    emb3 = emb.reshape(b, l2, 128)
    se, so, me, mo, parts = _emb_scores(emb3, w, 128)
    we, wo = _weights_apply(
        se, so, me, mo, parts, gamma[0::2], gamma[1::2], beta[0::2],
        beta[1::2], float(b), 128)
    return _pool_tc(emb3, we, wo, 128)


def kernel(code_seqs, query_seqs, code_table, code_w, code_gamma, code_beta,
           query_table, query_w, query_gamma, query_beta):
    pq = _encode(query_seqs, query_table, query_w, query_gamma, query_beta,
                 4000)
    pc = _encode(code_seqs, code_table, code_w, code_gamma, code_beta, 8000)
    return _similarity(pq, pc)
